# baseline (speedup 1.0000x reference)
import functools

import numpy as np
import jax
import jax.numpy as jnp
from jax import lax
from jax.experimental import pallas as pl
from jax.experimental.pallas import tpu as pltpu
from jax.experimental.pallas import tpu_sc as plsc

_N_ROWS = 16384
_SEQ_LEN = 200
_NA = 25

_NC = 2
_NS = 16
_NW = _NC * _NS

_ROWS_PER_W = _N_ROWS // _NW
_CHUNK_ROWS = 4
_NBUF = 4
_N_CHUNKS = _ROWS_PER_W // _CHUNK_ROWS
_CHUNK_POS = _CHUNK_ROWS * _SEQ_LEN
_CHUNK_WORDS = _CHUNK_POS * _NA
_N_GRP = _CHUNK_POS // 16

_KS0 = np.uint32(0)
_KS1 = np.uint32(42)
_KS2 = np.uint32(0x1BD11BDA ^ 42)
_ROTS = ((13, 15, 26, 6), (17, 29, 16, 24))
_INJECT = (
    (_KS1, np.uint32(_KS2 + np.uint32(1))),
    (_KS2, np.uint32(_KS0 + np.uint32(2))),
    (_KS0, np.uint32(_KS1 + np.uint32(3))),
    (_KS1, np.uint32(_KS2 + np.uint32(4))),
    (_KS2, np.uint32(_KS0 + np.uint32(5))),
)


def _threefry_bits(g):
  x1 = g + _KS1
  x0 = x1
  first = True
  for grp in range(5):
    for r in _ROTS[grp % 2]:
      if first:
        first = False
      else:
        x0 = x0 + x1
      x1 = ((x1 << np.uint32(r)) | (x1 >> np.uint32(32 - r))) ^ x0
    a, b = _INJECT[grp]
    x0 = x0 + a
    x1 = x1 + b
  return x0 ^ x1


def _uniform_from_g(g):
  bits = _threefry_bits(g)
  return lax.bitcast_convert_type(
      (bits >> np.uint32(9)) | np.uint32(0x3F800000), jnp.float32) - 1.0


def _sc_body(seq_hbm, out_hbm, *refs):
  seq_bufs = refs[:_NBUF]
  stage_bufs = refs[_NBUF:2 * _NBUF]
  unk_v, sem_seq, sem_out = refs[2 * _NBUF:]

  wid = lax.axis_index("s") * _NC + lax.axis_index("c")
  w_pos_base = wid * _ROWS_PER_W * _SEQ_LEN
  lanes = lax.iota(jnp.int32, 16)

  def seq_copy(c, b):
    return pltpu.make_async_copy(
        seq_hbm.at[pl.ds(w_pos_base + c * _CHUNK_POS, _CHUNK_POS)],
        seq_bufs[b], sem_seq.at[b])

  def out_copy(c, b):
    return pltpu.make_async_copy(
        stage_bufs[b],
        out_hbm.at[pl.ds((w_pos_base + c * _CHUNK_POS) * _NA, _CHUNK_WORDS)],
        sem_out.at[b])

  for b in range(_NBUF):
    seq_copy(b, b).start()

  def process(c, b):
    stage = stage_bufs[b]
    seqc = seq_bufs[b]
    @pl.when(c >= _NBUF)
    def _():
      out_copy(c - _NBUF, b).wait()

    seq_copy(c, b).wait()

    @plsc.parallel_loop(0, _N_GRP, unroll=2)
    def _(gidx):
      pos = gidx * 16 + lanes
      s = seqc[pl.ds(gidx * 16, 16)]
      base = pos * _NA
      for k in range(_NA):
        val = jnp.where(s == k, jnp.float32(1.0), jnp.float32(0.0))
        plsc.store_scatter(stage, [base + k], val)

    def compact_body(gidx, cnt):
      pos = gidx * 16 + lanes
      s = seqc[pl.ds(gidx * 16, 16)]
      m = s == 24
      plsc.store_compressed(unk_v.at[pl.ds(cnt, 16)], pos, mask=m)
      return cnt + jnp.sum(jnp.where(m, jnp.int32(1), jnp.int32(0)))

    cnt = lax.fori_loop(0, _N_GRP, compact_body, jnp.int32(0))

    g_chunk_base = (w_pos_base + c * _CHUNK_POS) * _NA

    def hash_body(h, _):
      valid = h * 16 + lanes < cnt
      posc = unk_v[pl.ds(h * 16, 16)]
      base = posc * _NA
      gbase = (g_chunk_base + posc * _NA).astype(jnp.uint32)

      def k_body(j, ssum):
        k0 = j * 5
        for t in range(5):
          k = k0 + t
          u = _uniform_from_g(gbase + k.astype(jnp.uint32))
          plsc.store_scatter(stage, [base + k], u, mask=valid)
          ssum = ssum + u
        return ssum

      ssum = lax.fori_loop(0, 5, k_body, jnp.zeros((16,), jnp.float32))
      inv = 1.0 / ssum

      def k_body2(j, _):
        k0 = j * 5
        for t in range(5):
          k = k0 + t
          u = plsc.load_gather(stage, [base + k], mask=valid)
          plsc.store_scatter(stage, [base + k], u * inv, mask=valid)
        return 0

      lax.fori_loop(0, 5, k_body2, jnp.int32(0))
      return 0

    nh = (cnt + 15) >> 4
    lax.fori_loop(0, nh, hash_body, jnp.int32(0))

    out_copy(c, b).start()
    @pl.when(c + _NBUF < _N_CHUNKS)
    def _():
      seq_copy(c + _NBUF, b).start()

  def outer(cb, carry):
    for b in range(_NBUF):
      process(cb * _NBUF + b, b)
    return carry

  lax.fori_loop(0, _N_CHUNKS // _NBUF, outer, jnp.int32(0))

  for b in range(_NBUF):
    out_copy(_N_CHUNKS - _NBUF + b, b).wait()


@jax.jit
def kernel(seq):
  seq_flat = seq.reshape(_N_ROWS * _SEQ_LEN)
  mesh = plsc.VectorSubcoreMesh(
      core_axis_name="c", subcore_axis_name="s", num_cores=_NC,
      num_subcores=_NS)
  scratch = (
      [pltpu.VMEM((_CHUNK_POS,), jnp.int32) for _ in range(_NBUF)] +
      [pltpu.VMEM((_CHUNK_WORDS,), jnp.float32) for _ in range(_NBUF)] + [
          pltpu.VMEM((_CHUNK_POS,), jnp.int32),
          pltpu.SemaphoreType.DMA((_NBUF,)),
          pltpu.SemaphoreType.DMA((_NBUF,)),
      ])
  out = pl.kernel(
      _sc_body,
      out_type=jax.ShapeDtypeStruct((_N_ROWS * _SEQ_LEN * _NA,), jnp.float32),
      mesh=mesh,
      compiler_params=pltpu.CompilerParams(needs_layout_passes=False),
      scratch_types=scratch,
  )(seq_flat)
  return out.reshape(_N_ROWS, _SEQ_LEN, _NA)
